# Initial kernel scaffold; baseline (speedup 1.0000x reference)
import functools

import jax
import jax.numpy as jnp
from jax import lax
from jax.experimental import pallas as pl
from jax.experimental.pallas import tpu as pltpu
from jax.experimental.pallas import tpu_sc as plsc

_N = 10000
_F = 128
_E = 320000
_TILES = 16
_CORES = 2
_NP = 10240
_ROWS_PT = _NP // _TILES
_CH = 128
_CHUNKS = 79
_EPT = _CH * _CHUNKS
_EPAD = _EPT * _TILES * _CORES
_DEGW = 16

_mesh = plsc.VectorSubcoreMesh(core_axis_name="c", subcore_axis_name="s")


def _zero_ref(ref, rows, cols):
    zv = jnp.zeros((16,), jnp.float32)

    def row_body(r, _):
        def col_body(k, _2):
            ref[r, pl.ds(k * 16, 16)] = zv
            return 0

        return lax.fori_loop(0, cols // 16, col_body, 0)

    lax.fori_loop(0, rows, row_body, 0)


def _fill_ones(ref, rows, cols):
    ov = jnp.ones((16,), jnp.float32)

    def row_body(r, _):
        def col_body(k, _2):
            ref[r, pl.ds(k * 16, 16)] = ov
            return 0

        return lax.fori_loop(0, cols // 16, col_body, 0)

    lax.fori_loop(0, rows, row_body, 0)


def _seg_sum_body(with_deg, *refs):
    if with_deg:
        (table, srcm, dstm, out, outd, sidx, didx, rows, zbuf, obuf, acc,
         accd, sem) = refs
    else:
        (table, srcm, dstm, out, sidx, didx, rows, zbuf, acc, sem) = refs
    c = lax.axis_index("c")
    s = lax.axis_index("s")

    _zero_ref(zbuf, 128, _F)
    for i in range(_ROWS_PT // 128):
        pltpu.sync_copy(zbuf, acc.at[pl.ds(s * _ROWS_PT + i * 128, 128)])
    if with_deg:
        _fill_ones(obuf, _CH, _DEGW)
        for i in range(_ROWS_PT // 128):
            pltpu.sync_copy(zbuf.at[:, pl.ds(0, _DEGW)],
                            accd.at[pl.ds(s * _ROWS_PT + i * 128, 128)])
    plsc.subcore_barrier()

    row0 = (c * _TILES + s) * _CHUNKS
    pltpu.sync_copy(srcm.at[pl.ds(row0, _CHUNKS)], sidx)
    pltpu.sync_copy(dstm.at[pl.ds(row0, _CHUNKS)], didx)

    def chunk(j, _):
        pltpu.async_copy(table.at[sidx.at[j]], rows, sem).wait()
        pltpu.sync_copy(rows, acc.at[didx.at[j]], add=True)
        if with_deg:
            pltpu.sync_copy(obuf, accd.at[didx.at[j]], add=True)
        return 0

    lax.fori_loop(0, _CHUNKS, chunk, 0)
    plsc.subcore_barrier()

    pltpu.sync_copy(acc.at[pl.ds(s * _ROWS_PT, _ROWS_PT)],
                    out.at[pl.ds(c * _NP + s * _ROWS_PT, _ROWS_PT)])
    if with_deg:
        pltpu.sync_copy(accd.at[pl.ds(s * _ROWS_PT, _ROWS_PT)],
                        outd.at[pl.ds(c * _NP + s * _ROWS_PT, _ROWS_PT)])


_seg_sum = pl.kernel(
    functools.partial(_seg_sum_body, False),
    out_type=(jax.ShapeDtypeStruct((_CORES * _NP, _F), jnp.float32),),
    mesh=_mesh,
    scratch_types=(
        pltpu.VMEM((_CHUNKS, _CH), jnp.int32),
        pltpu.VMEM((_CHUNKS, _CH), jnp.int32),
        pltpu.VMEM((_CH, _F), jnp.float32),
        pltpu.VMEM((128, _F), jnp.float32),
        pltpu.VMEM_SHARED((_NP, _F), jnp.float32),
        pltpu.SemaphoreType.DMA,
    ),
)

_seg_sum_deg = pl.kernel(
    functools.partial(_seg_sum_body, True),
    out_type=(jax.ShapeDtypeStruct((_CORES * _NP, _F), jnp.float32),
              jax.ShapeDtypeStruct((_CORES * _NP, _DEGW), jnp.float32)),
    mesh=_mesh,
    scratch_types=(
        pltpu.VMEM((_CHUNKS, _CH), jnp.int32),
        pltpu.VMEM((_CHUNKS, _CH), jnp.int32),
        pltpu.VMEM((_CH, _F), jnp.float32),
        pltpu.VMEM((128, _F), jnp.float32),
        pltpu.VMEM((_CH, _DEGW), jnp.float32),
        pltpu.VMEM_SHARED((_NP, _F), jnp.float32),
        pltpu.VMEM_SHARED((_NP, _DEGW), jnp.float32),
        pltpu.SemaphoreType.DMA,
    ),
)


def _row_mask(n_rows):
    i = lax.broadcasted_iota(jnp.int32, (n_rows, 1), 0)
    return (i < _N).astype(jnp.float32)


def _prep_body(x_ref, xh_ref, xsq_ref):
    x = x_ref[...]
    norm = jnp.sqrt(jnp.sum(x * x, axis=1, keepdims=True))
    xh = x / jnp.maximum(norm, 1e-8)
    pad = jnp.zeros((_NP - _N, _F), jnp.float32)
    xhp = jnp.concatenate([xh, pad], axis=0)
    xh_ref[...] = xhp
    xsq_ref[...] = xhp * xhp


def _prep(x):
    return pl.pallas_call(
        _prep_body,
        out_shape=(jax.ShapeDtypeStruct((_NP, _F), jnp.float32),
                   jax.ShapeDtypeStruct((_NP, _F), jnp.float32)),
    )(x)


def _colstats(v):
    m = jnp.mean(v, axis=0, keepdims=True)
    var = jnp.sum((v - m) * (v - m), axis=0, keepdims=True) / (v.shape[0] - 1)
    s = jnp.maximum(jnp.sqrt(var), 1e-8)
    return m, s


def _gate_body(x_ref, xh_ref, p1_ref, p2_ref, pd_ref, gW1_ref, gb1_ref,
               gW2_ref, gb2_ref, faW1_ref, fab1_ref, faW2_ref, fab2_ref,
               h0_ref):
    x = x_ref[...]
    xh = xh_ref[pl.ds(0, _N), :]
    s1 = p1_ref[pl.ds(0, _N), :] + p1_ref[pl.ds(_NP, _N), :]
    s2 = p2_ref[pl.ds(0, _N), :] + p2_ref[pl.ds(_NP, _N), :]
    deg = pd_ref[pl.ds(0, _N), pl.ds(0, 8)] + pd_ref[pl.ds(_NP, _N), pl.ds(0, 8)]
    deg = deg[:, 0:1]
    agg = deg * xh * xh - 2.0 * xh * s1 + s2
    r_normal = agg / (deg + 1e-12)
    r_flip = 2.0 - r_normal

    xm, xs = _colstats(x)
    xn = (x - xm) / xs
    g1 = jnp.maximum(
        jnp.dot(xn, gW1_ref[...], preferred_element_type=jnp.float32)
        + gb1_ref[...], 0.0)
    gates = jax.nn.sigmoid(
        jnp.dot(g1, gW2_ref[...], preferred_element_type=jnp.float32)
        + gb2_ref[...])

    rm, rs = _colstats(r_normal)
    rn = (r_normal - rm) / rs
    rf = (r_flip - rm) / rs
    z = gates * rn + (1.0 - gates) * rf
    zm, zs = _colstats(z)
    en = (z - zm) / zs
    a1 = jnp.maximum(
        jnp.dot(en, faW1_ref[...], preferred_element_type=jnp.float32)
        + fab1_ref[...], 0.0)
    attn = jax.nn.sigmoid(
        jnp.dot(a1, faW2_ref[...], preferred_element_type=jnp.float32)
        + fab2_ref[...])
    h0 = en * attn
    pad = jnp.zeros((_NP - _N, _F), jnp.float32)
    h0_ref[...] = jnp.concatenate([h0, pad], axis=0)


def _gate(x, xhp, p1, p2, pd, gW1, gb1, gW2, gb2, faW1, fab1, faW2, fab2):
    return pl.pallas_call(
        _gate_body,
        out_shape=jax.ShapeDtypeStruct((_NP, _F), jnp.float32),
    )(x, xhp, p1, p2, pd, gW1, gb1, gW2, gb2, faW1, fab1, faW2, fab2)


def _sage_body(h_ref, pn_ref, pd_ref, Ws_ref, Wn_ref, b_ref, out_ref):
    h = h_ref[...]
    nsum = pn_ref[pl.ds(0, _NP), :] + pn_ref[pl.ds(_NP, _NP), :]
    deg = pd_ref[pl.ds(0, _NP), pl.ds(0, 8)] + pd_ref[pl.ds(_NP, _NP), pl.ds(0, 8)]
    deg_c = jnp.maximum(deg[:, 0:1], 1.0)
    neigh = nsum / deg_c
    out = jnp.maximum(
        jnp.dot(h, Ws_ref[...], preferred_element_type=jnp.float32)
        + jnp.dot(neigh, Wn_ref[...], preferred_element_type=jnp.float32)
        + b_ref[...], 0.0)
    out_ref[...] = out * _row_mask(_NP)


def _sage(h, pn, pd, Ws, Wn, b):
    return pl.pallas_call(
        _sage_body,
        out_shape=jax.ShapeDtypeStruct((_NP, _F), jnp.float32),
    )(h, pn, pd, Ws, Wn, b)


def _final_body(h_ref, pn_ref, pd_ref, W3s_ref, W3n_ref, cb3_ref, Wc_ref,
                bc_ref, out_ref):
    h = h_ref[pl.ds(0, _N), :]
    nsum = pn_ref[pl.ds(0, _N), :] + pn_ref[pl.ds(_NP, _N), :]
    deg = pd_ref[pl.ds(0, _N), pl.ds(0, 8)] + pd_ref[pl.ds(_NP, _N), pl.ds(0, 8)]
    deg_c = jnp.maximum(deg[:, 0:1], 1.0)
    neigh = nsum / deg_c
    h3 = jnp.maximum(
        jnp.dot(h, W3s_ref[...], preferred_element_type=jnp.float32)
        + jnp.dot(neigh, W3n_ref[...], preferred_element_type=jnp.float32)
        + cb3_ref[...], 0.0)
    out_ref[...] = (jnp.dot(h3, Wc_ref[...], preferred_element_type=jnp.float32)
                    + bc_ref[...])


def _final(h, pn, pd, W3s, W3n, cb3, Wc, bc):
    return pl.pallas_call(
        _final_body,
        out_shape=jax.ShapeDtypeStruct((_N, 40), jnp.float32),
    )(h, pn, pd, W3s, W3n, cb3, Wc, bc)


def kernel(features, edge_index, gW1, gb1, gW2, gb2, faW1, fab1, faW2, fab2,
           W1s, W1n, cb1, W2s, W2n, cb2, W3s, W3n, cb3, Wc, bc):
    src = edge_index[0]
    dst = edge_index[1]
    padn = _EPAD - _E
    padv = jnp.full((padn,), _N, jnp.int32)
    srcm = jnp.concatenate([src, padv]).reshape(_EPAD // _CH, _CH)
    dstm = jnp.concatenate([dst, padv]).reshape(_EPAD // _CH, _CH)

    xhp, xsqp = _prep(features)
    p1, pdeg = _seg_sum_deg(xhp, srcm, dstm)
    (p2,) = _seg_sum(xsqp, srcm, dstm)
    h0 = _gate(features, xhp, p1, p2, pdeg,
               gW1, gb1, gW2, gb2, faW1, fab1, faW2, fab2)
    (p3,) = _seg_sum(h0, srcm, dstm)
    h1 = _sage(h0, p3, pdeg, W1s, W1n, cb1)
    (p4,) = _seg_sum(h1, srcm, dstm)
    h2 = _sage(h1, p4, pdeg, W2s, W2n, cb2)
    (p5,) = _seg_sum(h2, srcm, dstm)
    return _final(h2, p5, pdeg, W3s, W3n, cb3, Wc, bc)


# speedup vs baseline: 2.6301x; 2.6301x over previous
import functools

import jax
import jax.numpy as jnp
from jax import lax
from jax.experimental import pallas as pl
from jax.experimental.pallas import tpu as pltpu
from jax.experimental.pallas import tpu_sc as plsc

_N = 10000
_F = 128
_E = 320000
_TILES = 16
_CORES = 2
_NP = 10112
_ROWS_PT = _NP // _TILES
_CH = 128
_CHUNKS = 80
_EPAD = _CH * _CHUNKS * _TILES * _CORES


def _zero_ref(ref, rows, cols):
    zv = jnp.zeros((16,), jnp.float32)

    def row_body(r, _):
        def col_body(k, _2):
            ref[r, pl.ds(k * 16, 16)] = zv
            return 0

        return lax.fori_loop(0, cols // 16, col_body, 0)

    lax.fori_loop(0, rows, row_body, 0)


def _fill_ones(ref, rows, cols):
    ov = jnp.ones((16,), jnp.float32)

    def row_body(r, _):
        def col_body(k, _2):
            ref[r, pl.ds(k * 16, 16)] = ov
            return 0

        return lax.fori_loop(0, cols // 16, col_body, 0)

    lax.fori_loop(0, rows, row_body, 0)


def _zero_acc_slice(buf, acc, s):
    base = s * _ROWS_PT
    for i in range(4):
        pltpu.sync_copy(buf, acc.at[pl.ds(base + i * 128, 128)])
    pltpu.sync_copy(buf.at[pl.ds(0, _ROWS_PT - 512)],
                    acc.at[pl.ds(base + 512, _ROWS_PT - 512)])


def _seg_sum_body(table, srcm, dstm, out, sidx, didx, rows, acc, sem):
    c = lax.axis_index("c")
    s = lax.axis_index("s")

    _zero_ref(rows, _CH, _F)
    _zero_acc_slice(rows, acc, s)
    plsc.subcore_barrier()

    row0 = (c * _TILES + s) * _CHUNKS
    pltpu.sync_copy(srcm.at[pl.ds(row0, _CHUNKS)], sidx)
    pltpu.sync_copy(dstm.at[pl.ds(row0, _CHUNKS)], didx)

    def chunk(j, _):
        pltpu.async_copy(table.at[sidx.at[j]], rows, sem).wait()
        pltpu.sync_copy(rows, acc.at[didx.at[j]], add=True)
        return 0

    lax.fori_loop(0, _CHUNKS, chunk, 0)
    plsc.subcore_barrier()

    pltpu.sync_copy(acc.at[pl.ds(s * _ROWS_PT, _ROWS_PT)],
                    out.at[pl.ds(c * _NP + s * _ROWS_PT, _ROWS_PT)])


def _deg_body(dstm, out, didx, obuf, acc):
    c = lax.axis_index("c")
    s = lax.axis_index("s")

    _zero_ref(obuf, _CH, _F)
    _zero_acc_slice(obuf, acc, s)
    _fill_ones(obuf, _CH, _F)
    plsc.subcore_barrier()

    row0 = (c * _TILES + s) * _CHUNKS
    pltpu.sync_copy(dstm.at[pl.ds(row0, _CHUNKS)], didx)

    def chunk(j, _):
        pltpu.sync_copy(obuf, acc.at[didx.at[j]], add=True)
        return 0

    lax.fori_loop(0, _CHUNKS, chunk, 0)
    plsc.subcore_barrier()

    pltpu.sync_copy(acc.at[pl.ds(s * _ROWS_PT, _ROWS_PT)],
                    out.at[pl.ds(c * _NP + s * _ROWS_PT, _ROWS_PT)])


@functools.cache
def _get_seg_sum():
    mesh = plsc.VectorSubcoreMesh(core_axis_name="c", subcore_axis_name="s")
    return pl.kernel(
        _seg_sum_body,
        out_type=(jax.ShapeDtypeStruct((_CORES * _NP, _F), jnp.float32),),
        mesh=mesh,
        scratch_types=(
            pltpu.VMEM((_CHUNKS, _CH), jnp.int32),
            pltpu.VMEM((_CHUNKS, _CH), jnp.int32),
            pltpu.VMEM((_CH, _F), jnp.float32),
            pltpu.VMEM_SHARED((_NP, _F), jnp.float32),
            pltpu.SemaphoreType.DMA,
        ),
    )


@functools.cache
def _get_deg():
    mesh = plsc.VectorSubcoreMesh(core_axis_name="c", subcore_axis_name="s")
    return pl.kernel(
        _deg_body,
        out_type=(jax.ShapeDtypeStruct((_CORES * _NP, _F), jnp.float32),),
        mesh=mesh,
        scratch_types=(
            pltpu.VMEM((_CHUNKS, _CH), jnp.int32),
            pltpu.VMEM((_CH, _F), jnp.float32),
            pltpu.VMEM_SHARED((_NP, _F), jnp.float32),
        ),
    )


def _psum(p):
    return p[0:_N] + p[_NP:_NP + _N]


def _prep_body(x_ref, xh_ref, xsq_ref):
    x = x_ref[...]
    norm = jnp.sqrt(jnp.sum(x * x, axis=1, keepdims=True))
    xh = x / jnp.maximum(norm, 1e-8)
    pad = jnp.zeros((_NP - _N, _F), jnp.float32)
    xhp = jnp.concatenate([xh, pad], axis=0)
    xh_ref[...] = xhp
    xsq_ref[...] = xhp * xhp


def _prep(x):
    sds = jax.ShapeDtypeStruct((_NP, _F), jnp.float32)
    return pl.pallas_call(_prep_body, out_shape=(sds, sds))(x)


def _colstats(v):
    m = jnp.mean(v, axis=0, keepdims=True)
    var = jnp.sum((v - m) * (v - m), axis=0, keepdims=True) / (v.shape[0] - 1)
    s = jnp.maximum(jnp.sqrt(var), 1e-8)
    return m, s


def _gate_body(x_ref, xh_ref, p1_ref, p2_ref, pd_ref, gW1_ref, gb1_ref,
               gW2_ref, gb2_ref, faW1_ref, fab1_ref, faW2_ref, fab2_ref,
               h0_ref):
    x = x_ref[...]
    xh = xh_ref[pl.ds(0, _N), :]
    s1 = _psum(p1_ref[...])
    s2 = _psum(p2_ref[...])
    deg = _psum(pd_ref[...])[:, 0:1]
    agg = deg * xh * xh - 2.0 * xh * s1 + s2
    r_normal = agg / (deg + 1e-12)
    r_flip = 2.0 - r_normal

    xm, xs = _colstats(x)
    xn = (x - xm) / xs
    g1 = jnp.maximum(
        jnp.dot(xn, gW1_ref[...], preferred_element_type=jnp.float32)
        + gb1_ref[...], 0.0)
    gates = jax.nn.sigmoid(
        jnp.dot(g1, gW2_ref[...], preferred_element_type=jnp.float32)
        + gb2_ref[...])

    rm, rs = _colstats(r_normal)
    rn = (r_normal - rm) / rs
    rf = (r_flip - rm) / rs
    z = gates * rn + (1.0 - gates) * rf
    zm, zs = _colstats(z)
    en = (z - zm) / zs
    a1 = jnp.maximum(
        jnp.dot(en, faW1_ref[...], preferred_element_type=jnp.float32)
        + fab1_ref[...], 0.0)
    attn = jax.nn.sigmoid(
        jnp.dot(a1, faW2_ref[...], preferred_element_type=jnp.float32)
        + fab2_ref[...])
    h0 = en * attn
    pad = jnp.zeros((_NP - _N, _F), jnp.float32)
    h0_ref[...] = jnp.concatenate([h0, pad], axis=0)


def _gate(x, xhp, p1, p2, pd, gW1, gb1, gW2, gb2, faW1, fab1, faW2, fab2):
    return pl.pallas_call(
        _gate_body,
        out_shape=jax.ShapeDtypeStruct((_NP, _F), jnp.float32),
    )(x, xhp, p1, p2, pd, gW1, gb1, gW2, gb2, faW1, fab1, faW2, fab2)


def _sage_body(h_ref, pn_ref, pd_ref, Ws_ref, Wn_ref, b_ref, out_ref):
    h = h_ref[pl.ds(0, _N), :]
    nsum = _psum(pn_ref[...])
    deg_c = jnp.maximum(_psum(pd_ref[...])[:, 0:1], 1.0)
    neigh = nsum / deg_c
    out = jnp.maximum(
        jnp.dot(h, Ws_ref[...], preferred_element_type=jnp.float32)
        + jnp.dot(neigh, Wn_ref[...], preferred_element_type=jnp.float32)
        + b_ref[...], 0.0)
    pad = jnp.zeros((_NP - _N, _F), jnp.float32)
    out_ref[...] = jnp.concatenate([out, pad], axis=0)


def _sage(h, pn, pd, Ws, Wn, b):
    return pl.pallas_call(
        _sage_body,
        out_shape=jax.ShapeDtypeStruct((_NP, _F), jnp.float32),
    )(h, pn, pd, Ws, Wn, b)


def _final_body(h_ref, pn_ref, pd_ref, W3s_ref, W3n_ref, cb3_ref, Wc_ref,
                bc_ref, out_ref):
    h = h_ref[pl.ds(0, _N), :]
    nsum = _psum(pn_ref[...])
    deg_c = jnp.maximum(_psum(pd_ref[...])[:, 0:1], 1.0)
    neigh = nsum / deg_c
    h3 = jnp.maximum(
        jnp.dot(h, W3s_ref[...], preferred_element_type=jnp.float32)
        + jnp.dot(neigh, W3n_ref[...], preferred_element_type=jnp.float32)
        + cb3_ref[...], 0.0)
    out_ref[...] = (jnp.dot(h3, Wc_ref[...], preferred_element_type=jnp.float32)
                    + bc_ref[...])


def _final(h, pn, pd, W3s, W3n, cb3, Wc, bc):
    return pl.pallas_call(
        _final_body,
        out_shape=jax.ShapeDtypeStruct((_N, 40), jnp.float32),
    )(h, pn, pd, W3s, W3n, cb3, Wc, bc)


def kernel(features, edge_index, gW1, gb1, gW2, gb2, faW1, fab1, faW2, fab2,
           W1s, W1n, cb1, W2s, W2n, cb2, W3s, W3n, cb3, Wc, bc):
    src = edge_index[0]
    dst = edge_index[1]
    padn = _EPAD - _E
    padv = jnp.full((padn,), _N, jnp.int32)
    srcm = jnp.concatenate([src, padv]).reshape(_EPAD // _CH, _CH)
    dstm = jnp.concatenate([dst, padv]).reshape(_EPAD // _CH, _CH)

    seg_sum = _get_seg_sum()
    deg_pass = _get_deg()

    xhp, xsqp = _prep(features)
    (pdeg,) = deg_pass(dstm)
    (p1,) = seg_sum(xhp, srcm, dstm)
    (p2,) = seg_sum(xsqp, srcm, dstm)
    h0 = _gate(features, xhp, p1, p2, pdeg,
               gW1, gb1, gW2, gb2, faW1, fab1, faW2, fab2)
    (p3,) = seg_sum(h0, srcm, dstm)
    h1 = _sage(h0, p3, pdeg, W1s, W1n, cb1)
    (p4,) = seg_sum(h1, srcm, dstm)
    h2 = _sage(h1, p4, pdeg, W2s, W2n, cb2)
    (p5,) = seg_sum(h2, srcm, dstm)
    return _final(h2, p5, pdeg, W3s, W3n, cb3, Wc, bc)


# speedup vs baseline: 7.2188x; 2.7447x over previous
import functools

import jax
import jax.numpy as jnp
from jax import lax
from jax.experimental import pallas as pl
from jax.experimental.pallas import tpu as pltpu
from jax.experimental.pallas import tpu_sc as plsc

_N = 10000
_F = 128
_E = 320000
_TILES = 16
_CORES = 2
_NP = 10112
_ROWS_PT = _NP // _TILES
_CH = 128
_CHUNKS = 80
_HB = 40
_EPAD = _CH * _CHUNKS * _TILES * _CORES


def _fill_ones(ref, rows, cols):
    ov = jnp.ones((16,), jnp.float32)

    def row_body(r, _):
        def col_body(k, _2):
            ref[r, pl.ds(k * 16, 16)] = ov
            return 0

        return lax.fori_loop(0, cols // 16, col_body, 0)

    lax.fori_loop(0, rows, row_body, 0)


def _zero_acc_slice(table, acc, s):
    base = s * _ROWS_PT
    for i in range(6):
        pltpu.sync_copy(table.at[pl.ds(_N, 96)],
                        acc.at[pl.ds(base + i * 96, 96)])
    pltpu.sync_copy(table.at[pl.ds(_N, _ROWS_PT - 576)],
                    acc.at[pl.ds(base + 576, _ROWS_PT - 576)])


def _seg_sum_body(table, srcm, dstm, out, sidx, didx, rows0, rows1, acc,
                  gs0, gs1, ss0, ss1):
    c = lax.axis_index("c")
    s = lax.axis_index("s")

    _zero_acc_slice(table, acc, s)
    plsc.subcore_barrier()

    row0 = (c * _TILES + s) * _CHUNKS
    for half in range(_CHUNKS // _HB):
        pltpu.sync_copy(srcm.at[pl.ds(row0 + half * _HB, _HB)], sidx)
        pltpu.sync_copy(dstm.at[pl.ds(row0 + half * _HB, _HB)], didx)

        pltpu.async_copy(table.at[sidx.at[0]], rows0, gs0)
        pltpu.async_copy(table.at[sidx.at[1]], rows1, gs1)

        def pair(i, _):
            j = 2 * i
            pltpu.make_async_copy(table.at[sidx.at[j]], rows0, gs0).wait()
            pltpu.async_copy(rows0, acc.at[didx.at[j]], ss0, add=True)
            pltpu.make_async_copy(table.at[sidx.at[j + 1]], rows1, gs1).wait()
            pltpu.async_copy(rows1, acc.at[didx.at[j + 1]], ss1, add=True)
            pltpu.make_async_copy(rows0, acc.at[didx.at[j]], ss0).wait()
            pltpu.async_copy(table.at[sidx.at[j + 2]], rows0, gs0)
            pltpu.make_async_copy(rows1, acc.at[didx.at[j + 1]], ss1).wait()
            pltpu.async_copy(table.at[sidx.at[j + 3]], rows1, gs1)
            return 0

        lax.fori_loop(0, _HB // 2 - 1, pair, 0)

        j = _HB - 2
        pltpu.make_async_copy(table.at[sidx.at[j]], rows0, gs0).wait()
        pltpu.async_copy(rows0, acc.at[didx.at[j]], ss0, add=True)
        pltpu.make_async_copy(table.at[sidx.at[j + 1]], rows1, gs1).wait()
        pltpu.async_copy(rows1, acc.at[didx.at[j + 1]], ss1, add=True)
        pltpu.make_async_copy(rows0, acc.at[didx.at[j]], ss0).wait()
        pltpu.make_async_copy(rows1, acc.at[didx.at[j + 1]], ss1).wait()

    plsc.subcore_barrier()
    pltpu.sync_copy(acc.at[pl.ds(s * _ROWS_PT, _ROWS_PT)],
                    out.at[pl.ds(c * _NP + s * _ROWS_PT, _ROWS_PT)])


def _deg_body(ztable, dstm, out, didx, obuf, acc, ss):
    c = lax.axis_index("c")
    s = lax.axis_index("s")

    _zero_acc_slice(ztable, acc, s)
    _fill_ones(obuf, _CH, _F)
    plsc.subcore_barrier()

    row0 = (c * _TILES + s) * _CHUNKS
    pltpu.sync_copy(dstm.at[pl.ds(row0, _CHUNKS)], didx)

    def fire(j, _):
        pltpu.async_copy(obuf, acc.at[didx.at[j]], ss, add=True)
        return 0

    lax.fori_loop(0, _CHUNKS, fire, 0)

    def drain(j, _):
        pltpu.make_async_copy(obuf, acc.at[didx.at[j]], ss).wait()
        return 0

    lax.fori_loop(0, _CHUNKS, drain, 0)
    plsc.subcore_barrier()

    pltpu.sync_copy(acc.at[pl.ds(s * _ROWS_PT, _ROWS_PT)],
                    out.at[pl.ds(c * _NP + s * _ROWS_PT, _ROWS_PT)])


@functools.cache
def _get_seg_sum():
    mesh = plsc.VectorSubcoreMesh(core_axis_name="c", subcore_axis_name="s")
    return pl.kernel(
        _seg_sum_body,
        out_type=(jax.ShapeDtypeStruct((_CORES * _NP, _F), jnp.float32),),
        mesh=mesh,
        scratch_types=(
            pltpu.VMEM((_HB, _CH), jnp.int32),
            pltpu.VMEM((_HB, _CH), jnp.int32),
            pltpu.VMEM((_CH, _F), jnp.float32),
            pltpu.VMEM((_CH, _F), jnp.float32),
            pltpu.VMEM_SHARED((_NP, _F), jnp.float32),
            pltpu.SemaphoreType.DMA,
            pltpu.SemaphoreType.DMA,
            pltpu.SemaphoreType.DMA,
            pltpu.SemaphoreType.DMA,
        ),
    )


@functools.cache
def _get_deg():
    mesh = plsc.VectorSubcoreMesh(core_axis_name="c", subcore_axis_name="s")
    return pl.kernel(
        _deg_body,
        out_type=(jax.ShapeDtypeStruct((_CORES * _NP, _F), jnp.float32),),
        mesh=mesh,
        scratch_types=(
            pltpu.VMEM((_CHUNKS, _CH), jnp.int32),
            pltpu.VMEM((_CH, _F), jnp.float32),
            pltpu.VMEM_SHARED((_NP, _F), jnp.float32),
            pltpu.SemaphoreType.DMA,
        ),
    )


def _psum(p):
    return p[0:_N] + p[_NP:_NP + _N]


def _prep_body(x_ref, xh_ref, xsq_ref):
    x = x_ref[...]
    norm = jnp.sqrt(jnp.sum(x * x, axis=1, keepdims=True))
    xh = x / jnp.maximum(norm, 1e-8)
    pad = jnp.zeros((_NP - _N, _F), jnp.float32)
    xhp = jnp.concatenate([xh, pad], axis=0)
    xh_ref[...] = xhp
    xsq_ref[...] = xhp * xhp


def _prep(x):
    sds = jax.ShapeDtypeStruct((_NP, _F), jnp.float32)
    return pl.pallas_call(_prep_body, out_shape=(sds, sds))(x)


def _colstats(v):
    m = jnp.mean(v, axis=0, keepdims=True)
    var = jnp.sum((v - m) * (v - m), axis=0, keepdims=True) / (v.shape[0] - 1)
    s = jnp.maximum(jnp.sqrt(var), 1e-8)
    return m, s


def _gate_body(x_ref, xh_ref, p1_ref, p2_ref, pd_ref, gW1_ref, gb1_ref,
               gW2_ref, gb2_ref, faW1_ref, fab1_ref, faW2_ref, fab2_ref,
               h0_ref):
    x = x_ref[...]
    xh = xh_ref[pl.ds(0, _N), :]
    s1 = _psum(p1_ref[...])
    s2 = _psum(p2_ref[...])
    deg = _psum(pd_ref[...])[:, 0:1]
    agg = deg * xh * xh - 2.0 * xh * s1 + s2
    r_normal = agg / (deg + 1e-12)
    r_flip = 2.0 - r_normal

    xm, xs = _colstats(x)
    xn = (x - xm) / xs
    g1 = jnp.maximum(
        jnp.dot(xn, gW1_ref[...], preferred_element_type=jnp.float32)
        + gb1_ref[...], 0.0)
    gates = jax.nn.sigmoid(
        jnp.dot(g1, gW2_ref[...], preferred_element_type=jnp.float32)
        + gb2_ref[...])

    rm, rs = _colstats(r_normal)
    rn = (r_normal - rm) / rs
    rf = (r_flip - rm) / rs
    z = gates * rn + (1.0 - gates) * rf
    zm, zs = _colstats(z)
    en = (z - zm) / zs
    a1 = jnp.maximum(
        jnp.dot(en, faW1_ref[...], preferred_element_type=jnp.float32)
        + fab1_ref[...], 0.0)
    attn = jax.nn.sigmoid(
        jnp.dot(a1, faW2_ref[...], preferred_element_type=jnp.float32)
        + fab2_ref[...])
    h0 = en * attn
    pad = jnp.zeros((_NP - _N, _F), jnp.float32)
    h0_ref[...] = jnp.concatenate([h0, pad], axis=0)


def _gate(x, xhp, p1, p2, pd, gW1, gb1, gW2, gb2, faW1, fab1, faW2, fab2):
    return pl.pallas_call(
        _gate_body,
        out_shape=jax.ShapeDtypeStruct((_NP, _F), jnp.float32),
    )(x, xhp, p1, p2, pd, gW1, gb1, gW2, gb2, faW1, fab1, faW2, fab2)


def _sage_body(h_ref, pn_ref, pd_ref, Ws_ref, Wn_ref, b_ref, out_ref):
    h = h_ref[pl.ds(0, _N), :]
    nsum = _psum(pn_ref[...])
    deg_c = jnp.maximum(_psum(pd_ref[...])[:, 0:1], 1.0)
    neigh = nsum / deg_c
    out = jnp.maximum(
        jnp.dot(h, Ws_ref[...], preferred_element_type=jnp.float32)
        + jnp.dot(neigh, Wn_ref[...], preferred_element_type=jnp.float32)
        + b_ref[...], 0.0)
    pad = jnp.zeros((_NP - _N, _F), jnp.float32)
    out_ref[...] = jnp.concatenate([out, pad], axis=0)


def _sage(h, pn, pd, Ws, Wn, b):
    return pl.pallas_call(
        _sage_body,
        out_shape=jax.ShapeDtypeStruct((_NP, _F), jnp.float32),
    )(h, pn, pd, Ws, Wn, b)


def _final_body(h_ref, pn_ref, pd_ref, W3s_ref, W3n_ref, cb3_ref, Wc_ref,
                bc_ref, out_ref):
    h = h_ref[pl.ds(0, _N), :]
    nsum = _psum(pn_ref[...])
    deg_c = jnp.maximum(_psum(pd_ref[...])[:, 0:1], 1.0)
    neigh = nsum / deg_c
    h3 = jnp.maximum(
        jnp.dot(h, W3s_ref[...], preferred_element_type=jnp.float32)
        + jnp.dot(neigh, W3n_ref[...], preferred_element_type=jnp.float32)
        + cb3_ref[...], 0.0)
    out_ref[...] = (jnp.dot(h3, Wc_ref[...], preferred_element_type=jnp.float32)
                    + bc_ref[...])


def _final(h, pn, pd, W3s, W3n, cb3, Wc, bc):
    return pl.pallas_call(
        _final_body,
        out_shape=jax.ShapeDtypeStruct((_N, 40), jnp.float32),
    )(h, pn, pd, W3s, W3n, cb3, Wc, bc)


def kernel(features, edge_index, gW1, gb1, gW2, gb2, faW1, fab1, faW2, fab2,
           W1s, W1n, cb1, W2s, W2n, cb2, W3s, W3n, cb3, Wc, bc):
    src = edge_index[0]
    dst = edge_index[1]
    padn = _EPAD - _E
    padv = _N + (jnp.arange(padn, dtype=jnp.int32) % (_NP - _N))
    srcm = jnp.concatenate([src, padv]).reshape(_EPAD // _CH, _CH)
    dstm = jnp.concatenate([dst, padv]).reshape(_EPAD // _CH, _CH)

    seg_sum = _get_seg_sum()
    deg_pass = _get_deg()

    xhp, xsqp = _prep(features)
    (pdeg,) = deg_pass(xhp, dstm)
    (p1,) = seg_sum(xhp, srcm, dstm)
    (p2,) = seg_sum(xsqp, srcm, dstm)
    h0 = _gate(features, xhp, p1, p2, pdeg,
               gW1, gb1, gW2, gb2, faW1, fab1, faW2, fab2)
    (p3,) = seg_sum(h0, srcm, dstm)
    h1 = _sage(h0, p3, pdeg, W1s, W1n, cb1)
    (p4,) = seg_sum(h1, srcm, dstm)
    h2 = _sage(h1, p4, pdeg, W2s, W2n, cb2)
    (p5,) = seg_sum(h2, srcm, dstm)
    return _final(h2, p5, pdeg, W3s, W3n, cb3, Wc, bc)


# speedup vs baseline: 7.3389x; 1.0166x over previous
import functools

import jax
import jax.numpy as jnp
from jax import lax
from jax.experimental import pallas as pl
from jax.experimental.pallas import tpu as pltpu
from jax.experimental.pallas import tpu_sc as plsc

_N = 10000
_F = 128
_E = 320000
_TILES = 16
_CORES = 2
_NP = 10112
_ROWS_PT = _NP // _TILES
_CH = 128
_CHUNKS = 80
_HB = 40
_EPAD = _CH * _CHUNKS * _TILES * _CORES


def _fill_ones(ref, rows, cols):
    ov = jnp.ones((16,), jnp.float32)

    def row_body(r, _):
        def col_body(k, _2):
            ref[r, pl.ds(k * 16, 16)] = ov
            return 0

        return lax.fori_loop(0, cols // 16, col_body, 0)

    lax.fori_loop(0, rows, row_body, 0)


def _zero_acc_slice(table, acc, s):
    base = s * _ROWS_PT
    for i in range(6):
        pltpu.sync_copy(table.at[pl.ds(_N, 96)],
                        acc.at[pl.ds(base + i * 96, 96)])
    pltpu.sync_copy(table.at[pl.ds(_N, _ROWS_PT - 576)],
                    acc.at[pl.ds(base + 576, _ROWS_PT - 576)])


def _seg_sum_body(table, srcm, dstm, out, sidx, didx, arena, acc,
                  sg0, sg1, sg2, sg3, ss0, ss1):
    c = lax.axis_index("c")
    s = lax.axis_index("s")

    _zero_acc_slice(table, acc, s)
    plsc.subcore_barrier()

    half0 = arena.at[pl.ds(0, _CH)]
    half1 = arena.at[pl.ds(_CH, _CH)]
    q = [arena.at[pl.ds(k * 64, 64)] for k in range(4)]
    sg = [sg0, sg1, sg2, sg3]

    def fire_gather(j, h):
        pltpu.async_copy(table.at[sidx.at[j, pl.ds(0, 64)]], q[2 * h],
                         sg[2 * h])
        pltpu.async_copy(table.at[sidx.at[j, pl.ds(64, 64)]], q[2 * h + 1],
                         sg[2 * h + 1])

    def wait_gather(j, h):
        pltpu.make_async_copy(table.at[sidx.at[j, pl.ds(0, 64)]], q[2 * h],
                              sg[2 * h]).wait()
        pltpu.make_async_copy(table.at[sidx.at[j, pl.ds(64, 64)]],
                              q[2 * h + 1], sg[2 * h + 1]).wait()

    row0 = (c * _TILES + s) * _CHUNKS
    for half in range(_CHUNKS // _HB):
        pltpu.sync_copy(srcm.at[pl.ds(row0 + half * _HB, _HB)], sidx)
        pltpu.sync_copy(dstm.at[pl.ds(row0 + half * _HB, _HB)], didx)

        fire_gather(0, 0)
        fire_gather(1, 1)

        def pair(i, _):
            j = 2 * i
            wait_gather(j, 0)
            pltpu.async_copy(half0, acc.at[didx.at[j]], ss0, add=True)
            wait_gather(j + 1, 1)
            pltpu.async_copy(half1, acc.at[didx.at[j + 1]], ss1, add=True)
            pltpu.make_async_copy(half0, acc.at[didx.at[j]], ss0).wait()
            fire_gather(j + 2, 0)
            pltpu.make_async_copy(half1, acc.at[didx.at[j + 1]], ss1).wait()
            fire_gather(j + 3, 1)
            return 0

        lax.fori_loop(0, _HB // 2 - 1, pair, 0)

        j = _HB - 2
        wait_gather(j, 0)
        pltpu.async_copy(half0, acc.at[didx.at[j]], ss0, add=True)
        wait_gather(j + 1, 1)
        pltpu.async_copy(half1, acc.at[didx.at[j + 1]], ss1, add=True)
        pltpu.make_async_copy(half0, acc.at[didx.at[j]], ss0).wait()
        pltpu.make_async_copy(half1, acc.at[didx.at[j + 1]], ss1).wait()

    plsc.subcore_barrier()
    pltpu.sync_copy(acc.at[pl.ds(s * _ROWS_PT, _ROWS_PT)],
                    out.at[pl.ds(c * _NP + s * _ROWS_PT, _ROWS_PT)])


def _deg_body(ztable, dstm, out, didx, obuf, acc, ss):
    c = lax.axis_index("c")
    s = lax.axis_index("s")

    _zero_acc_slice(ztable, acc, s)
    _fill_ones(obuf, _CH, _F)
    plsc.subcore_barrier()

    row0 = (c * _TILES + s) * _CHUNKS
    pltpu.sync_copy(dstm.at[pl.ds(row0, _CHUNKS)], didx)

    def fire(j, _):
        pltpu.async_copy(obuf, acc.at[didx.at[j]], ss, add=True)
        return 0

    lax.fori_loop(0, _CHUNKS, fire, 0)

    def drain(j, _):
        pltpu.make_async_copy(obuf, acc.at[didx.at[j]], ss).wait()
        return 0

    lax.fori_loop(0, _CHUNKS, drain, 0)
    plsc.subcore_barrier()

    pltpu.sync_copy(acc.at[pl.ds(s * _ROWS_PT, _ROWS_PT)],
                    out.at[pl.ds(c * _NP + s * _ROWS_PT, _ROWS_PT)])


@functools.cache
def _get_seg_sum():
    mesh = plsc.VectorSubcoreMesh(core_axis_name="c", subcore_axis_name="s")
    return pl.kernel(
        _seg_sum_body,
        out_type=(jax.ShapeDtypeStruct((_CORES * _NP, _F), jnp.float32),),
        mesh=mesh,
        scratch_types=(
            pltpu.VMEM((_HB, _CH), jnp.int32),
            pltpu.VMEM((_HB, _CH), jnp.int32),
            pltpu.VMEM((2 * _CH, _F), jnp.float32),
            pltpu.VMEM_SHARED((_NP, _F), jnp.float32),
            pltpu.SemaphoreType.DMA,
            pltpu.SemaphoreType.DMA,
            pltpu.SemaphoreType.DMA,
            pltpu.SemaphoreType.DMA,
            pltpu.SemaphoreType.DMA,
            pltpu.SemaphoreType.DMA,
        ),
    )


@functools.cache
def _get_deg():
    mesh = plsc.VectorSubcoreMesh(core_axis_name="c", subcore_axis_name="s")
    return pl.kernel(
        _deg_body,
        out_type=(jax.ShapeDtypeStruct((_CORES * _NP, _F), jnp.float32),),
        mesh=mesh,
        scratch_types=(
            pltpu.VMEM((_CHUNKS, _CH), jnp.int32),
            pltpu.VMEM((_CH, _F), jnp.float32),
            pltpu.VMEM_SHARED((_NP, _F), jnp.float32),
            pltpu.SemaphoreType.DMA,
        ),
    )


def _psum(p):
    return p[0:_N] + p[_NP:_NP + _N]


def _prep_body(x_ref, xh_ref, xsq_ref):
    x = x_ref[...]
    norm = jnp.sqrt(jnp.sum(x * x, axis=1, keepdims=True))
    xh = x / jnp.maximum(norm, 1e-8)
    pad = jnp.zeros((_NP - _N, _F), jnp.float32)
    xhp = jnp.concatenate([xh, pad], axis=0)
    xh_ref[...] = xhp
    xsq_ref[...] = xhp * xhp


def _prep(x):
    sds = jax.ShapeDtypeStruct((_NP, _F), jnp.float32)
    return pl.pallas_call(_prep_body, out_shape=(sds, sds))(x)


def _colstats(v):
    m = jnp.mean(v, axis=0, keepdims=True)
    var = jnp.sum((v - m) * (v - m), axis=0, keepdims=True) / (v.shape[0] - 1)
    s = jnp.maximum(jnp.sqrt(var), 1e-8)
    return m, s


def _gate_body(x_ref, xh_ref, p1_ref, p2_ref, pd_ref, gW1_ref, gb1_ref,
               gW2_ref, gb2_ref, faW1_ref, fab1_ref, faW2_ref, fab2_ref,
               h0_ref):
    x = x_ref[...]
    xh = xh_ref[pl.ds(0, _N), :]
    s1 = _psum(p1_ref[...])
    s2 = _psum(p2_ref[...])
    deg = _psum(pd_ref[...])[:, 0:1]
    agg = deg * xh * xh - 2.0 * xh * s1 + s2
    r_normal = agg / (deg + 1e-12)
    r_flip = 2.0 - r_normal

    xm, xs = _colstats(x)
    xn = (x - xm) / xs
    g1 = jnp.maximum(
        jnp.dot(xn, gW1_ref[...], preferred_element_type=jnp.float32)
        + gb1_ref[...], 0.0)
    gates = jax.nn.sigmoid(
        jnp.dot(g1, gW2_ref[...], preferred_element_type=jnp.float32)
        + gb2_ref[...])

    rm, rs = _colstats(r_normal)
    rn = (r_normal - rm) / rs
    rf = (r_flip - rm) / rs
    z = gates * rn + (1.0 - gates) * rf
    zm, zs = _colstats(z)
    en = (z - zm) / zs
    a1 = jnp.maximum(
        jnp.dot(en, faW1_ref[...], preferred_element_type=jnp.float32)
        + fab1_ref[...], 0.0)
    attn = jax.nn.sigmoid(
        jnp.dot(a1, faW2_ref[...], preferred_element_type=jnp.float32)
        + fab2_ref[...])
    h0 = en * attn
    pad = jnp.zeros((_NP - _N, _F), jnp.float32)
    h0_ref[...] = jnp.concatenate([h0, pad], axis=0)


def _gate(x, xhp, p1, p2, pd, gW1, gb1, gW2, gb2, faW1, fab1, faW2, fab2):
    return pl.pallas_call(
        _gate_body,
        out_shape=jax.ShapeDtypeStruct((_NP, _F), jnp.float32),
    )(x, xhp, p1, p2, pd, gW1, gb1, gW2, gb2, faW1, fab1, faW2, fab2)


def _sage_body(h_ref, pn_ref, pd_ref, Ws_ref, Wn_ref, b_ref, out_ref):
    h = h_ref[pl.ds(0, _N), :]
    nsum = _psum(pn_ref[...])
    deg_c = jnp.maximum(_psum(pd_ref[...])[:, 0:1], 1.0)
    neigh = nsum / deg_c
    out = jnp.maximum(
        jnp.dot(h, Ws_ref[...], preferred_element_type=jnp.float32)
        + jnp.dot(neigh, Wn_ref[...], preferred_element_type=jnp.float32)
        + b_ref[...], 0.0)
    pad = jnp.zeros((_NP - _N, _F), jnp.float32)
    out_ref[...] = jnp.concatenate([out, pad], axis=0)


def _sage(h, pn, pd, Ws, Wn, b):
    return pl.pallas_call(
        _sage_body,
        out_shape=jax.ShapeDtypeStruct((_NP, _F), jnp.float32),
    )(h, pn, pd, Ws, Wn, b)


def _final_body(h_ref, pn_ref, pd_ref, W3s_ref, W3n_ref, cb3_ref, Wc_ref,
                bc_ref, out_ref):
    h = h_ref[pl.ds(0, _N), :]
    nsum = _psum(pn_ref[...])
    deg_c = jnp.maximum(_psum(pd_ref[...])[:, 0:1], 1.0)
    neigh = nsum / deg_c
    h3 = jnp.maximum(
        jnp.dot(h, W3s_ref[...], preferred_element_type=jnp.float32)
        + jnp.dot(neigh, W3n_ref[...], preferred_element_type=jnp.float32)
        + cb3_ref[...], 0.0)
    out_ref[...] = (jnp.dot(h3, Wc_ref[...], preferred_element_type=jnp.float32)
                    + bc_ref[...])


def _final(h, pn, pd, W3s, W3n, cb3, Wc, bc):
    return pl.pallas_call(
        _final_body,
        out_shape=jax.ShapeDtypeStruct((_N, 40), jnp.float32),
    )(h, pn, pd, W3s, W3n, cb3, Wc, bc)


def kernel(features, edge_index, gW1, gb1, gW2, gb2, faW1, fab1, faW2, fab2,
           W1s, W1n, cb1, W2s, W2n, cb2, W3s, W3n, cb3, Wc, bc):
    src = edge_index[0]
    dst = edge_index[1]
    padn = _EPAD - _E
    padv = _N + (jnp.arange(padn, dtype=jnp.int32) % (_NP - _N))
    srcm = jnp.concatenate([src, padv]).reshape(_EPAD // _CH, _CH)
    dstm = jnp.concatenate([dst, padv]).reshape(_EPAD // _CH, _CH)

    seg_sum = _get_seg_sum()
    deg_pass = _get_deg()

    xhp, xsqp = _prep(features)
    (pdeg,) = deg_pass(xhp, dstm)
    (p1,) = seg_sum(xhp, srcm, dstm)
    (p2,) = seg_sum(xsqp, srcm, dstm)
    h0 = _gate(features, xhp, p1, p2, pdeg,
               gW1, gb1, gW2, gb2, faW1, fab1, faW2, fab2)
    (p3,) = seg_sum(h0, srcm, dstm)
    h1 = _sage(h0, p3, pdeg, W1s, W1n, cb1)
    (p4,) = seg_sum(h1, srcm, dstm)
    h2 = _sage(h1, p4, pdeg, W2s, W2n, cb2)
    (p5,) = seg_sum(h2, srcm, dstm)
    return _final(h2, p5, pdeg, W3s, W3n, cb3, Wc, bc)
